# baseline (speedup 1.0000x reference)
import functools

import jax
import jax.numpy as jnp
from jax import lax
from jax.experimental import pallas as pl
from jax.experimental.pallas import tpu as pltpu
from jax.experimental.pallas import tpu_sc as plsc

BS = 4096
N_VARS = 26
IN_LEN = 200
N_CAT = 10
ROW = N_VARS * IN_LEN
NC, NS = 2, 16
NW = NC * NS
ROWS_PER_W = BS // NW
CHUNK = 4
N_CHUNKS = ROWS_PER_W // CHUNK
CWORDS = CHUNK * ROW
PREF = N_CAT * IN_LEN
GWORDS = CHUNK * PREF
NBUF = 3
TSLICES = list(range(0, IN_LEN - 32, 16))
TLAST = (IN_LEN - 24, IN_LEN - 16)
FMAGIC = jnp.float32(2.0 ** 23)
FBIAS = jnp.int32(0x4B000000)


def _sc_embed(x_flat, table):
    mesh = plsc.VectorSubcoreMesh(core_axis_name="c", subcore_axis_name="s")

    @functools.partial(
        pl.kernel,
        mesh=mesh,
        out_type=jax.ShapeDtypeStruct((BS * ROW,), jnp.float32),
        scratch_types=[
            pltpu.VMEM((N_CAT * 1000,), jnp.float32),
            pltpu.VMEM((CWORDS,), jnp.float32),
            pltpu.VMEM((CWORDS,), jnp.float32),
            pltpu.VMEM((CWORDS,), jnp.float32),
            pltpu.VMEM((GWORDS,), jnp.float32),
            pltpu.VMEM((GWORDS,), jnp.float32),
            pltpu.VMEM((GWORDS,), jnp.float32),
            pltpu.SemaphoreType.DMA,
            pltpu.SemaphoreType.DMA,
            pltpu.SemaphoreType.DMA,
            pltpu.SemaphoreType.DMA,
            pltpu.SemaphoreType.DMA,
            pltpu.SemaphoreType.DMA,
        ],
        compiler_params=pltpu.CompilerParams(needs_layout_passes=False),
    )
    def k(x_hbm, tab_hbm, out_hbm, tab_v, buf0, buf1, buf2, g0, g1, g2,
          si0, si1, si2, so0, so1, so2):
        wid = lax.axis_index("s") * NC + lax.axis_index("c")
        pltpu.sync_copy(tab_hbm, tab_v)
        base = wid * (ROWS_PER_W * ROW)
        bufs = (buf0, buf1, buf2)
        gs = (g0, g1, g2)
        sis, sos = (si0, si1, si2), (so0, so1, so2)

        def issue_in(ci, b):
            pltpu.async_copy(
                x_hbm.at[pl.ds(base + ci * CWORDS, CWORDS)], bufs[b], sis[b]
            )

        def issue_out(ci, b):
            for r in range(CHUNK):
                rowoff = base + ci * CWORDS + r * ROW
                pltpu.async_copy(
                    gs[b].at[pl.ds(r * PREF, PREF)],
                    out_hbm.at[pl.ds(rowoff, PREF)],
                    sos[b],
                )
                pltpu.async_copy(
                    bufs[b].at[pl.ds(r * ROW + PREF, ROW - PREF)],
                    out_hbm.at[pl.ds(rowoff + PREF, ROW - PREF)],
                    sos[b],
                )

        def wait_in(b):
            pltpu.make_async_copy(
                x_hbm.at[pl.ds(0, CWORDS)], bufs[b], sis[b]
            ).wait()

        def wait_out(b):
            pltpu.make_async_copy(
                bufs[b], out_hbm.at[pl.ds(0, CWORDS)], sos[b]
            ).wait()

        def compute(b):
            buf, g = bufs[b], gs[b]

            def rc_body(rc, c2):
                r = rc // N_CAT
                c = rc - r * N_CAT
                cb = r * ROW + c * IN_LEN
                gb = r * PREF + c * IN_LEN
                bias = FBIAS - c * 1000

                def do(t):
                    v = buf[pl.ds(cb + t, 16)] + FMAGIC
                    return plsc.load_gather(
                        tab_v, [plsc.bitcast(v, jnp.int32) - bias]
                    )

                for t in TSLICES + list(TLAST):
                    g[pl.ds(gb + t, 16)] = do(t)
                return c2

            lax.fori_loop(0, CHUNK * N_CAT, rc_body, 0)

        issue_in(0, 0)
        issue_in(1, 1)
        for ci in range(N_CHUNKS):
            b = ci % NBUF
            wait_in(b)
            compute(b)
            issue_out(ci, b)
            if ci + 2 < N_CHUNKS:
                nb = (ci + 2) % NBUF
                if ci >= 1:
                    wait_out(nb)
                issue_in(ci + 2, nb)
        for ci in range(N_CHUNKS - NBUF, N_CHUNKS):
            wait_out(ci % NBUF)

    return k(x_flat, table)


def kernel(x, W0, W1, W2, W3, W4, W5, W6, W7, W8, W9):
    table = jnp.concatenate(
        [W0, W1, W2, W3, W4, W5, W6, W7, W8, W9], axis=0
    )[:, 0]
    out = _sc_embed(x.reshape(BS * ROW), table)
    return out.reshape(BS, N_VARS, IN_LEN)


# speedup vs baseline: 1.7301x; 1.7301x over previous
import functools

import jax
import jax.numpy as jnp
from jax import lax
from jax.experimental import pallas as pl
from jax.experimental.pallas import tpu as pltpu
from jax.experimental.pallas import tpu_sc as plsc

BS = 4096
N_VARS = 26
IN_LEN = 200
N_CAT = 10
NC, NS = 2, 16
NW = NC * NS
ROWS_PER_W = BS // NW
CHUNK = 2
N_CHUNKS = ROWS_PER_W // CHUNK
NBUF = 4
TS0 = list(range(0, 128, 16))
TS1 = [128, 144, 160]
TLAST = (176, 184)
FMAGIC = jnp.float32(2.0 ** 23)
FBIAS = jnp.int32(0x4B000000)


def _sc_embed(x, table):
    mesh = plsc.VectorSubcoreMesh(core_axis_name="c", subcore_axis_name="s")

    @functools.partial(
        pl.kernel,
        mesh=mesh,
        out_type=jax.ShapeDtypeStruct((BS, N_VARS, IN_LEN), jnp.float32),
        scratch_types=[
            pltpu.VMEM((N_CAT * 1000,), jnp.float32),
            pltpu.VMEM((CHUNK, N_VARS, IN_LEN), jnp.float32),
            pltpu.VMEM((CHUNK, N_VARS, IN_LEN), jnp.float32),
            pltpu.VMEM((CHUNK, N_VARS, IN_LEN), jnp.float32),
            pltpu.VMEM((CHUNK, N_VARS, IN_LEN), jnp.float32),
            pltpu.SemaphoreType.DMA,
            pltpu.SemaphoreType.DMA,
            pltpu.SemaphoreType.DMA,
            pltpu.SemaphoreType.DMA,
            pltpu.SemaphoreType.DMA,
            pltpu.SemaphoreType.DMA,
            pltpu.SemaphoreType.DMA,
            pltpu.SemaphoreType.DMA,
        ],
        compiler_params=pltpu.CompilerParams(
            needs_layout_passes=False, use_tc_tiling_on_sc=True
        ),
    )
    def k(x_hbm, tab_hbm, out_hbm, tab_v, b0, b1, b2, b3,
          si0, si1, si2, si3, so0, so1, so2, so3):
        wid = lax.axis_index("s") * NC + lax.axis_index("c")
        pltpu.sync_copy(tab_hbm, tab_v)
        base = wid * ROWS_PER_W
        bufs = (b0, b1, b2, b3)
        sis, sos = (si0, si1, si2, si3), (so0, so1, so2, so3)

        def issue_in(ci, b):
            pltpu.async_copy(
                x_hbm.at[pl.ds(base + ci * CHUNK, CHUNK)], bufs[b], sis[b]
            )

        def issue_out(ci, b):
            pltpu.async_copy(
                bufs[b], out_hbm.at[pl.ds(base + ci * CHUNK, CHUNK)], sos[b]
            )

        def wait_in(b):
            pltpu.make_async_copy(
                x_hbm.at[pl.ds(0, CHUNK)], bufs[b], sis[b]
            ).wait()

        def wait_out(b):
            pltpu.make_async_copy(
                bufs[b], out_hbm.at[pl.ds(0, CHUNK)], sos[b]
            ).wait()

        def compute(b):
            buf = bufs[b]

            def row_body(r, c2):
                for c in range(N_CAT):
                    bias = FBIAS - c * 1000

                    def do(t):
                        v = buf[r, c, pl.ds(t, 16)] + FMAGIC
                        return plsc.load_gather(
                            tab_v, [plsc.bitcast(v, jnp.int32) - bias]
                        )

                    for t in TS0 + TS1:
                        buf[r, c, pl.ds(t, 16)] = do(t)
                    ga = do(TLAST[0])
                    gb = do(TLAST[1])
                    buf[r, c, pl.ds(TLAST[0], 16)] = ga
                    buf[r, c, pl.ds(TLAST[1], 16)] = gb
                return c2

            lax.fori_loop(0, CHUNK, row_body, 0)

        issue_in(0, 0)
        issue_in(1, 1)

        def step(j, carry):
            for p in range(NBUF):
                ci = NBUF * j + p
                q = (p + 2) % NBUF
                wait_in(p)
                compute(p)
                issue_out(ci, p)
                if p < 2:
                    pl.when(j != 0)(lambda q=q: wait_out(q))
                else:
                    wait_out(q)
                issue_in(lax.rem(ci + 2, N_CHUNKS), q)
            return carry

        lax.fori_loop(0, N_CHUNKS // NBUF, step, 0)

        wait_out(2)
        wait_out(3)
        wait_in(0)
        wait_in(1)

    return k(x, table)


def kernel(x, W0, W1, W2, W3, W4, W5, W6, W7, W8, W9):
    table = jnp.concatenate(
        [W0, W1, W2, W3, W4, W5, W6, W7, W8, W9], axis=0
    )[:, 0]
    return _sc_embed(x, table)


# speedup vs baseline: 1.7437x; 1.0079x over previous
import functools

import jax
import jax.numpy as jnp
from jax import lax
from jax.experimental import pallas as pl
from jax.experimental.pallas import tpu as pltpu
from jax.experimental.pallas import tpu_sc as plsc

BS = 4096
N_VARS = 26
IN_LEN = 200
N_CAT = 10
NC, NS = 2, 16
NW = NC * NS
ROWS_PER_W = BS // NW
CHUNK = 2
N_CHUNKS = ROWS_PER_W // CHUNK
NBUF = 4
TS0 = list(range(0, 128, 16))
TS1 = [128, 144, 160]
TLAST = (176, 184)
FMAGIC = jnp.float32(2.0 ** 23)
FBIAS = jnp.int32(0x4B000000)


def _sc_embed(x, table):
    mesh = plsc.VectorSubcoreMesh(core_axis_name="c", subcore_axis_name="s")

    @functools.partial(
        pl.kernel,
        mesh=mesh,
        out_type=jax.ShapeDtypeStruct((BS, N_VARS, IN_LEN), jnp.float32),
        scratch_types=[
            pltpu.VMEM((N_CAT * 1000,), jnp.float32),
            pltpu.VMEM((CHUNK, N_VARS, IN_LEN), jnp.float32),
            pltpu.VMEM((CHUNK, N_VARS, IN_LEN), jnp.float32),
            pltpu.VMEM((CHUNK, N_VARS, IN_LEN), jnp.float32),
            pltpu.VMEM((CHUNK, N_VARS, IN_LEN), jnp.float32),
            pltpu.SemaphoreType.DMA,
            pltpu.SemaphoreType.DMA,
            pltpu.SemaphoreType.DMA,
            pltpu.SemaphoreType.DMA,
            pltpu.SemaphoreType.DMA,
            pltpu.SemaphoreType.DMA,
            pltpu.SemaphoreType.DMA,
            pltpu.SemaphoreType.DMA,
        ],
        compiler_params=pltpu.CompilerParams(
            needs_layout_passes=False, use_tc_tiling_on_sc=True
        ),
    )
    def k(x_hbm, tab_hbm, out_hbm, tab_v, b0, b1, b2, b3,
          si0, si1, si2, si3, so0, so1, so2, so3):
        wid = lax.axis_index("s") * NC + lax.axis_index("c")
        pltpu.sync_copy(tab_hbm, tab_v)
        base = wid * ROWS_PER_W
        bufs = (b0, b1, b2, b3)
        sis, sos = (si0, si1, si2, si3), (so0, so1, so2, so3)

        def issue_in(ci, b):
            pltpu.async_copy(
                x_hbm.at[pl.ds(base + ci * CHUNK, CHUNK)], bufs[b], sis[b]
            )

        def issue_out(ci, b):
            pltpu.async_copy(
                bufs[b], out_hbm.at[pl.ds(base + ci * CHUNK, CHUNK)], sos[b]
            )

        def wait_in(b):
            pltpu.make_async_copy(
                x_hbm.at[pl.ds(0, CHUNK)], bufs[b], sis[b]
            ).wait()

        def wait_out(b):
            pltpu.make_async_copy(
                bufs[b], out_hbm.at[pl.ds(0, CHUNK)], sos[b]
            ).wait()

        def compute(b):
            buf = bufs[b]

            def rc_body(rc, c2):
                r = rc // N_CAT
                c = rc - r * N_CAT
                bias = FBIAS - c * 1000

                def do(t):
                    v = buf[r, c, pl.ds(t, 16)] + FMAGIC
                    return plsc.load_gather(
                        tab_v, [plsc.bitcast(v, jnp.int32) - bias]
                    )

                for t in TS0 + TS1:
                    buf[r, c, pl.ds(t, 16)] = do(t)
                ga = do(TLAST[0])
                gb = do(TLAST[1])
                buf[r, c, pl.ds(TLAST[0], 16)] = ga
                buf[r, c, pl.ds(TLAST[1], 16)] = gb
                return c2

            lax.fori_loop(0, CHUNK * N_CAT, rc_body, 0)

        issue_in(0, 0)
        issue_in(1, 1)

        def step(j, carry):
            for p in range(NBUF):
                ci = NBUF * j + p
                q = (p + 2) % NBUF
                wait_in(p)
                compute(p)
                issue_out(ci, p)
                if p < 2:
                    pl.when(j != 0)(lambda q=q: wait_out(q))
                else:
                    wait_out(q)
                issue_in(lax.rem(ci + 2, N_CHUNKS), q)
            return carry

        lax.fori_loop(0, N_CHUNKS // NBUF, step, 0)

        wait_out(2)
        wait_out(3)
        wait_in(0)
        wait_in(1)

    return k(x, table)


def kernel(x, W0, W1, W2, W3, W4, W5, W6, W7, W8, W9):
    table = jnp.concatenate(
        [W0, W1, W2, W3, W4, W5, W6, W7, W8, W9], axis=0
    )[:, 0]
    return _sc_embed(x, table)
